# Initial kernel scaffold; baseline (speedup 1.0000x reference)
import jax
import jax.numpy as jnp
from jax.experimental import pallas as pl


def kernel(pos, edge_index, atom_types):
    raise NotImplementedError("write your pallas kernel here")


# speedup vs baseline: 234.8757x; 234.8757x over previous
import functools

import numpy as np
import jax
import jax.numpy as jnp
from jax import lax
from jax.experimental import pallas as pl
from jax.experimental.pallas import tpu as pltpu
from jax.experimental.pallas import tpu_sc as plsc

N = 1000
NP = 1024
E = 4000
EC = E // 16
AC = NP // 16
NBLK = 8

_MV_T = np.full(16, 4.0, dtype=np.float32)
_MV_T[1] = 1.0; _MV_T[7] = 3.0; _MV_T[8] = 2.0; _MV_T[9] = 1.0
_LT = np.full((10, 10), 1.5, dtype=np.float32)
for (a, b), l in {(1, 6): 1.09, (6, 6): 1.54, (6, 7): 1.47, (6, 8): 1.43,
                  (6, 9): 1.35, (7, 7): 1.45, (7, 8): 1.40, (1, 7): 1.01,
                  (8, 8): 1.48, (1, 8): 0.96}.items():
    _LT[a, b] = l; _LT[b, a] = l
_LT_T = np.zeros(112, dtype=np.float32)
_LT_T[:100] = _LT.reshape(-1)
_VDW_T = np.full(16, 1.6, dtype=np.float32)
_VDW_T[1] = 1.2; _VDW_T[6] = 1.7; _VDW_T[7] = 1.55; _VDW_T[8] = 1.52; _VDW_T[9] = 1.47

_f32 = jnp.float32
_i32 = jnp.int32


def _rsqrt(x):
    xc = jnp.maximum(x, jnp.float32(1e-35))
    i = lax.bitcast_convert_type(xc, jnp.int32)
    i = jnp.int32(0x5F375A86) - lax.shift_right_logical(i, 1)
    y = lax.bitcast_convert_type(i, jnp.float32)
    for _ in range(3):
        y = y * (jnp.float32(1.5) - jnp.float32(0.5) * xc * y * y)
    return y


_sc_mesh = plsc.VectorSubcoreMesh(core_axis_name="c", subcore_axis_name="s")


@functools.partial(
    pl.kernel,
    out_type=(
        jax.ShapeDtypeStruct((NP,), _f32),
        jax.ShapeDtypeStruct((NP,), _f32),
        jax.ShapeDtypeStruct((NP,), _f32),
        jax.ShapeDtypeStruct((NP,), _f32),
        jax.ShapeDtypeStruct((16,), _f32),
        jax.ShapeDtypeStruct((16,), _f32),
    ),
    mesh=_sc_mesh,
    compiler_params=pltpu.CompilerParams(needs_layout_passes=False),
    scratch_types=[
        pltpu.VMEM((NP,), _f32),
        pltpu.VMEM((NP,), _f32),
        pltpu.VMEM((NP,), _f32),
        pltpu.VMEM((E,), _i32),
        pltpu.VMEM((E,), _i32),
        pltpu.VMEM((E,), _i32),
        pltpu.VMEM((E,), _i32),
        pltpu.VMEM((E,), _i32),
        pltpu.VMEM((E,), _f32),
        pltpu.VMEM((NP,), _i32),
        pltpu.VMEM((NP,), _f32),
        pltpu.VMEM((NP,), _f32),
        pltpu.VMEM((NP,), _f32),
        pltpu.VMEM((NP,), _f32),
        pltpu.VMEM((NP,), _f32),
        pltpu.VMEM((NP,), _f32),
        pltpu.VMEM((16,), _f32),
        pltpu.VMEM((112,), _f32),
        pltpu.VMEM((16,), _f32),
        pltpu.VMEM((16,), _f32),
    ],
)
def _sc_forward(px_h, py_h, pz_h, row_h, col_h, ord_h, at_h, mv_h, lt_h, vdw_h,
                ox_h, oy_h, oz_h, rad_h, l1_h, l2_h,
                vpx, vpy, vpz, vrow, vcol, vord, vrs, vcs, vef, vat, vcnt, vfac,
                vax, vay, vaz, vrad, vmv, vlt, vvdw, vtmp16):
    is_w0 = (lax.axis_index("c") == 0) & (lax.axis_index("s") == 0)

    @pl.when(is_w0)
    def _():
        pltpu.sync_copy(px_h, vpx)
        pltpu.sync_copy(py_h, vpy)
        pltpu.sync_copy(pz_h, vpz)
        pltpu.sync_copy(row_h, vrow)
        pltpu.sync_copy(col_h, vcol)
        pltpu.sync_copy(ord_h, vord)
        pltpu.sync_copy(at_h, vat)
        pltpu.sync_copy(mv_h, vmv)
        pltpu.sync_copy(lt_h, vlt)
        pltpu.sync_copy(vdw_h, vvdw)

        iota = lax.iota(_i32, 16)
        m0 = iota == 0
        mall = iota < 16
        zeros = jnp.zeros((16,), _f32)
        ones = jnp.ones((16,), _f32)

        def zinit(a, carry):
            sl = pl.ds(a * 16, 16)
            vcnt[sl] = zeros
            vax[sl] = zeros
            vay[sl] = zeros
            vaz[sl] = zeros
            return carry
        lax.fori_loop(0, AC, zinit, 0)

        def c1(k, carry):
            sl = pl.ds(k * 16, 16)
            plsc.addupdate_scatter(vcnt, [vrow[sl]], ones, mask=mall)
            ov = vord[sl]
            vrs[sl] = plsc.load_gather(vrow, [ov])
            vcs[sl] = plsc.load_gather(vcol, [ov])
            return carry
        lax.fori_loop(0, EC, c1, 0)

        def c2(a, l1acc):
            sl = pl.ds(a * 16, 16)
            atv = vat[sl]
            mvv = plsc.load_gather(vmv, [atv])
            cv = vcnt[sl]
            viol = cv - mvv
            m = cv > mvv
            l1acc = l1acc + jnp.where(m, viol * viol, jnp.float32(0.0))
            vfac[sl] = jnp.where(m, viol * jnp.float32(0.001), jnp.float32(0.0))
            vrad[sl] = plsc.load_gather(vvdw, [atv])
            return l1acc
        l1acc = lax.fori_loop(0, AC, c2, zeros)
        vtmp16[...] = l1acc
        pltpu.sync_copy(vtmp16, l1_h)

        def c3(k, carry):
            sl = pl.ds(k * 16, 16)
            vef[sl] = plsc.load_gather(vfac, [vrs[sl]])
            return carry
        lax.fori_loop(0, EC, c3, 0)

        def seq(k, carry):
            bk = jnp.zeros((16,), _i32) + k
            fv = plsc.load_gather(vef, [bk], mask=m0)
            iv = plsc.load_gather(vrs, [bk], mask=m0)
            jv = plsc.load_gather(vcs, [bk], mask=m0)
            xi = plsc.load_gather(vpx, [iv], mask=m0)
            yi = plsc.load_gather(vpy, [iv], mask=m0)
            zi = plsc.load_gather(vpz, [iv], mask=m0)
            xj = plsc.load_gather(vpx, [jv], mask=m0)
            yj = plsc.load_gather(vpy, [jv], mask=m0)
            zj = plsc.load_gather(vpz, [jv], mask=m0)
            dx = xi - xj
            dy = yi - yj
            dz = zi - zj
            d2 = dx * dx + dy * dy + dz * dz
            f = fv * _rsqrt(d2)
            plsc.store_scatter(vpx, [iv], xi + dx * f, mask=m0)
            plsc.store_scatter(vpy, [iv], yi + dy * f, mask=m0)
            plsc.store_scatter(vpz, [iv], zi + dz * f, mask=m0)
            return carry
        lax.fori_loop(0, E, seq, 0)

        def c4(k, l2acc):
            sl = pl.ds(k * 16, 16)
            rv = vrow[sl]
            cv = vcol[sl]
            xi = plsc.load_gather(vpx, [rv])
            yi = plsc.load_gather(vpy, [rv])
            zi = plsc.load_gather(vpz, [rv])
            xj = plsc.load_gather(vpx, [cv])
            yj = plsc.load_gather(vpy, [cv])
            zj = plsc.load_gather(vpz, [cv])
            bx = xi - xj
            by = yi - yj
            bz = zi - zj
            d2 = bx * bx + by * by + bz * bz
            cur = d2 * _rsqrt(d2)
            ati = plsc.load_gather(vat, [rv])
            atj = plsc.load_gather(vat, [cv])
            tv = plsc.load_gather(vlt, [ati * 10 + atj])
            le = cur - tv
            l2acc = l2acc + le * le
            rc = _rsqrt(cur + jnp.float32(1e-8))
            inv = rc * rc
            ratio = jnp.clip(tv * inv, jnp.float32(0.98), jnp.float32(1.02))
            g = (ratio - jnp.float32(1.0)) * jnp.float32(0.005)
            plsc.addupdate_scatter(vax, [rv], bx * g, mask=mall)
            plsc.addupdate_scatter(vay, [rv], by * g, mask=mall)
            plsc.addupdate_scatter(vaz, [rv], bz * g, mask=mall)
            plsc.addupdate_scatter(vax, [cv], -(bx * g), mask=mall)
            plsc.addupdate_scatter(vay, [cv], -(by * g), mask=mall)
            plsc.addupdate_scatter(vaz, [cv], -(bz * g), mask=mall)
            return l2acc
        l2acc = lax.fori_loop(0, EC, c4, zeros)
        vtmp16[...] = l2acc
        pltpu.sync_copy(vtmp16, l2_h)

        def c5(a, carry):
            sl = pl.ds(a * 16, 16)
            vpx[sl] = vpx[sl] + vax[sl]
            vpy[sl] = vpy[sl] + vay[sl]
            vpz[sl] = vpz[sl] + vaz[sl]
            return carry
        lax.fori_loop(0, AC, c5, 0)

        pltpu.sync_copy(vpx, ox_h)
        pltpu.sync_copy(vpy, oy_h)
        pltpu.sync_copy(vpz, oz_h)
        pltpu.sync_copy(vrad, rad_h)


def _tc_body(xr_ref, yr_ref, zr_ref, rr_ref, xt_ref, yt_ref, zt_ref, rt_ref,
             l1_ref, l2_ref, ox_ref, oy_ref, oz_ref, loss_ref):
    b = pl.program_id(0)
    xi = xt_ref[...]
    yi = yt_ref[...]
    zi = zt_ref[...]
    xj = xr_ref[...]
    yj = yr_ref[...]
    zj = zr_ref[...]
    dx = xi - xj
    dy = yi - yj
    dz = zi - zj
    d2 = dx * dx + dy * dy + dz * dz
    rowid = b * 128 + lax.broadcasted_iota(_i32, (128, NP), 0)
    colid = lax.broadcasted_iota(_i32, (128, NP), 1)
    diag = rowid == colid
    valid = jnp.logical_not(diag) & (rowid < N) & (colid < N)
    dist = jnp.sqrt(jnp.where(diag, jnp.float32(1.0), d2))
    md = (rt_ref[...] + rr_ref[...]) * jnp.float32(0.8)
    gap = md - dist
    pen = jnp.where(valid, jnp.maximum(gap, jnp.float32(0.0)), jnp.float32(0.0))
    l3p = jnp.float32(0.5) * jnp.sum(pen * pen)
    cmask = valid & (dist < md) & (dist > jnp.float32(1e-8))
    w = jnp.where(cmask, gap * jnp.float32(0.0025) / dist, jnp.float32(0.0))
    s = jnp.sum(w, axis=1, keepdims=True)
    tx = jnp.sum(w * xj, axis=1, keepdims=True)
    ty = jnp.sum(w * yj, axis=1, keepdims=True)
    tz = jnp.sum(w * zj, axis=1, keepdims=True)
    ox_ref[...] = xi + (xi * s - tx)
    oy_ref[...] = yi + (yi * s - ty)
    oz_ref[...] = zi + (zi * s - tz)

    @pl.when(b == 0)
    def _():
        loss_ref[...] = jnp.reshape(
            jnp.sum(l1_ref[...]) + jnp.sum(l2_ref[...]) * jnp.float32(1.0 / 4000.0),
            (1, 1))

    loss_ref[...] = loss_ref[...] + jnp.reshape(l3p, (1, 1))

    @pl.when(b == NBLK - 1)
    def _():
        loss_ref[...] = loss_ref[...] * jnp.float32(0.1)


_tc_clash = pl.pallas_call(
    _tc_body,
    grid=(NBLK,),
    in_specs=[
        pl.BlockSpec((1, NP), lambda b: (0, 0)),
        pl.BlockSpec((1, NP), lambda b: (0, 0)),
        pl.BlockSpec((1, NP), lambda b: (0, 0)),
        pl.BlockSpec((1, NP), lambda b: (0, 0)),
        pl.BlockSpec((128, 1), lambda b: (b, 0)),
        pl.BlockSpec((128, 1), lambda b: (b, 0)),
        pl.BlockSpec((128, 1), lambda b: (b, 0)),
        pl.BlockSpec((128, 1), lambda b: (b, 0)),
        pl.BlockSpec((1, 16), lambda b: (0, 0)),
        pl.BlockSpec((1, 16), lambda b: (0, 0)),
    ],
    out_specs=[
        pl.BlockSpec((128, 1), lambda b: (b, 0)),
        pl.BlockSpec((128, 1), lambda b: (b, 0)),
        pl.BlockSpec((128, 1), lambda b: (b, 0)),
        pl.BlockSpec((1, 1), lambda b: (0, 0)),
    ],
    out_shape=[
        jax.ShapeDtypeStruct((NP, 1), _f32),
        jax.ShapeDtypeStruct((NP, 1), _f32),
        jax.ShapeDtypeStruct((NP, 1), _f32),
        jax.ShapeDtypeStruct((1, 1), _f32),
    ],
)


def kernel(pos, edge_index, atom_types):
    row = edge_index[0]
    col = edge_index[1]
    order = jnp.argsort(row).astype(_i32)
    px = jnp.zeros((NP,), _f32).at[:N].set(pos[:, 0])
    py = jnp.zeros((NP,), _f32).at[:N].set(pos[:, 1])
    pz = jnp.zeros((NP,), _f32).at[:N].set(pos[:, 2])
    atp = jnp.zeros((NP,), _i32).at[:N].set(atom_types)
    ox, oy, oz, rad, l1, l2 = _sc_forward(
        px, py, pz, row, col, order, atp,
        jnp.asarray(_MV_T), jnp.asarray(_LT_T), jnp.asarray(_VDW_T))
    fx, fy, fz, loss = _tc_clash(
        ox.reshape(1, NP), oy.reshape(1, NP), oz.reshape(1, NP),
        rad.reshape(1, NP),
        ox.reshape(NP, 1), oy.reshape(NP, 1), oz.reshape(NP, 1),
        rad.reshape(NP, 1),
        l1.reshape(1, 16), l2.reshape(1, 16))
    pos_new = jnp.concatenate([fx, fy, fz], axis=1)[:N]
    return pos_new, loss[0, 0]


# speedup vs baseline: 291.9648x; 1.2431x over previous
import functools

import numpy as np
import jax
import jax.numpy as jnp
from jax import lax
from jax.experimental import pallas as pl
from jax.experimental.pallas import tpu as pltpu
from jax.experimental.pallas import tpu_sc as plsc

N = 1000
NP = 1024
E = 4000
EC = E // 16
AC = NP // 16
NBLK = 8

_MV_T = np.full(16, 4.0, dtype=np.float32)
_MV_T[1] = 1.0; _MV_T[7] = 3.0; _MV_T[8] = 2.0; _MV_T[9] = 1.0
_LT = np.full((10, 10), 1.5, dtype=np.float32)
for (a, b), l in {(1, 6): 1.09, (6, 6): 1.54, (6, 7): 1.47, (6, 8): 1.43,
                  (6, 9): 1.35, (7, 7): 1.45, (7, 8): 1.40, (1, 7): 1.01,
                  (8, 8): 1.48, (1, 8): 0.96}.items():
    _LT[a, b] = l; _LT[b, a] = l
_LT_T = np.zeros(112, dtype=np.float32)
_LT_T[:100] = _LT.reshape(-1)
_VDW_T = np.full(16, 1.6, dtype=np.float32)
_VDW_T[1] = 1.2; _VDW_T[6] = 1.7; _VDW_T[7] = 1.55; _VDW_T[8] = 1.52; _VDW_T[9] = 1.47

_f32 = jnp.float32
_i32 = jnp.int32


def _rsqrt(x):
    xc = jnp.maximum(x, jnp.float32(1e-35))
    i = lax.bitcast_convert_type(xc, jnp.int32)
    i = jnp.int32(0x5F375A86) - lax.shift_right_logical(i, 1)
    y = lax.bitcast_convert_type(i, jnp.float32)
    for _ in range(3):
        y = y * (jnp.float32(1.5) - jnp.float32(0.5) * xc * y * y)
    return y


_sc_mesh = plsc.VectorSubcoreMesh(core_axis_name="c", subcore_axis_name="s")


@functools.partial(
    pl.kernel,
    out_type=(
        jax.ShapeDtypeStruct((NP,), _f32),
        jax.ShapeDtypeStruct((NP,), _f32),
        jax.ShapeDtypeStruct((NP,), _f32),
        jax.ShapeDtypeStruct((NP,), _f32),
        jax.ShapeDtypeStruct((16,), _f32),
        jax.ShapeDtypeStruct((16,), _f32),
    ),
    mesh=_sc_mesh,
    compiler_params=pltpu.CompilerParams(needs_layout_passes=False),
    scratch_types=[
        pltpu.VMEM((NP,), _f32),
        pltpu.VMEM((NP,), _f32),
        pltpu.VMEM((NP,), _f32),
        pltpu.VMEM((E,), _i32),
        pltpu.VMEM((E,), _i32),
        pltpu.VMEM((E,), _i32),
        pltpu.VMEM((E + 16,), _i32),
        pltpu.VMEM((E + 16,), _i32),
        pltpu.VMEM((E + 16,), _f32),
        pltpu.VMEM((NP,), _i32),
        pltpu.VMEM((NP,), _f32),
        pltpu.VMEM((NP,), _f32),
        pltpu.VMEM((NP,), _f32),
        pltpu.VMEM((NP,), _f32),
        pltpu.VMEM((NP,), _f32),
        pltpu.VMEM((NP,), _f32),
        pltpu.VMEM((16,), _f32),
        pltpu.VMEM((112,), _f32),
        pltpu.VMEM((16,), _f32),
        pltpu.VMEM((16,), _f32),
    ],
)
def _sc_forward(px_h, py_h, pz_h, row_h, col_h, ord_h, at_h, mv_h, lt_h, vdw_h,
                ox_h, oy_h, oz_h, rad_h, l1_h, l2_h,
                vpx, vpy, vpz, vrow, vcol, vord, vrs, vcs, vef, vat, vcnt, vfac,
                vax, vay, vaz, vrad, vmv, vlt, vvdw, vtmp16):
    is_w0 = (lax.axis_index("c") == 0) & (lax.axis_index("s") == 0)

    @pl.when(is_w0)
    def _():
        pltpu.sync_copy(px_h, vpx)
        pltpu.sync_copy(py_h, vpy)
        pltpu.sync_copy(pz_h, vpz)
        pltpu.sync_copy(row_h, vrow)
        pltpu.sync_copy(col_h, vcol)
        pltpu.sync_copy(ord_h, vord)
        pltpu.sync_copy(at_h, vat)
        pltpu.sync_copy(mv_h, vmv)
        pltpu.sync_copy(lt_h, vlt)
        pltpu.sync_copy(vdw_h, vvdw)

        iota = lax.iota(_i32, 16)
        m0 = iota == 0
        mall = iota < 16
        zeros = jnp.zeros((16,), _f32)
        ones = jnp.ones((16,), _f32)

        def zinit(a, carry):
            sl = pl.ds(a * 16, 16)
            vcnt[sl] = zeros
            vax[sl] = zeros
            vay[sl] = zeros
            vaz[sl] = zeros
            return carry
        lax.fori_loop(0, AC, zinit, 0)

        def c1(k, carry):
            sl = pl.ds(k * 16, 16)
            plsc.addupdate_scatter(vcnt, [vrow[sl]], ones, mask=mall)
            return carry
        lax.fori_loop(0, EC, c1, 0)

        def c2(a, l1acc):
            sl = pl.ds(a * 16, 16)
            atv = vat[sl]
            mvv = plsc.load_gather(vmv, [atv])
            cv = vcnt[sl]
            viol = cv - mvv
            m = cv > mvv
            l1acc = l1acc + jnp.where(m, viol * viol, jnp.float32(0.0))
            vfac[sl] = jnp.where(m, viol * jnp.float32(0.001), jnp.float32(0.0))
            vrad[sl] = plsc.load_gather(vvdw, [atv])
            return l1acc
        l1acc = lax.fori_loop(0, AC, c2, zeros)
        vtmp16[...] = l1acc
        pltpu.sync_copy(vtmp16, l1_h)

        def c3(k, off):
            sl = pl.ds(k * 16, 16)
            ov = vord[sl]
            rv = plsc.load_gather(vrow, [ov])
            cv = plsc.load_gather(vcol, [ov])
            fv = plsc.load_gather(vfac, [rv])
            m = fv > jnp.float32(0.0)
            plsc.store_compressed(vrs.at[pl.ds(off, 16)], rv, mask=m)
            plsc.store_compressed(vcs.at[pl.ds(off, 16)], cv, mask=m)
            plsc.store_compressed(vef.at[pl.ds(off, 16)], fv, mask=m)
            return off + jnp.max(plsc.all_reduce_population_count(m))
        nkeep = lax.fori_loop(0, EC, c3, jnp.int32(0))

        def seq(k, carry):
            bk = jnp.zeros((16,), _i32) + k
            fv = plsc.load_gather(vef, [bk], mask=m0)
            iv = plsc.load_gather(vrs, [bk], mask=m0)
            jv = plsc.load_gather(vcs, [bk], mask=m0)
            xi = plsc.load_gather(vpx, [iv], mask=m0)
            yi = plsc.load_gather(vpy, [iv], mask=m0)
            zi = plsc.load_gather(vpz, [iv], mask=m0)
            xj = plsc.load_gather(vpx, [jv], mask=m0)
            yj = plsc.load_gather(vpy, [jv], mask=m0)
            zj = plsc.load_gather(vpz, [jv], mask=m0)
            dx = xi - xj
            dy = yi - yj
            dz = zi - zj
            d2 = dx * dx + dy * dy + dz * dz
            f = fv * _rsqrt(d2)
            plsc.store_scatter(vpx, [iv], xi + dx * f, mask=m0)
            plsc.store_scatter(vpy, [iv], yi + dy * f, mask=m0)
            plsc.store_scatter(vpz, [iv], zi + dz * f, mask=m0)
            return carry
        lax.fori_loop(0, nkeep, seq, 0)

        def c4(k, l2acc):
            sl = pl.ds(k * 16, 16)
            rv = vrow[sl]
            cv = vcol[sl]
            xi = plsc.load_gather(vpx, [rv])
            yi = plsc.load_gather(vpy, [rv])
            zi = plsc.load_gather(vpz, [rv])
            xj = plsc.load_gather(vpx, [cv])
            yj = plsc.load_gather(vpy, [cv])
            zj = plsc.load_gather(vpz, [cv])
            bx = xi - xj
            by = yi - yj
            bz = zi - zj
            d2 = bx * bx + by * by + bz * bz
            cur = d2 * _rsqrt(d2)
            ati = plsc.load_gather(vat, [rv])
            atj = plsc.load_gather(vat, [cv])
            tv = plsc.load_gather(vlt, [ati * 10 + atj])
            le = cur - tv
            l2acc = l2acc + le * le
            rc = _rsqrt(cur + jnp.float32(1e-8))
            inv = rc * rc
            ratio = jnp.clip(tv * inv, jnp.float32(0.98), jnp.float32(1.02))
            g = (ratio - jnp.float32(1.0)) * jnp.float32(0.005)
            plsc.addupdate_scatter(vax, [rv], bx * g, mask=mall)
            plsc.addupdate_scatter(vay, [rv], by * g, mask=mall)
            plsc.addupdate_scatter(vaz, [rv], bz * g, mask=mall)
            plsc.addupdate_scatter(vax, [cv], -(bx * g), mask=mall)
            plsc.addupdate_scatter(vay, [cv], -(by * g), mask=mall)
            plsc.addupdate_scatter(vaz, [cv], -(bz * g), mask=mall)
            return l2acc
        l2acc = lax.fori_loop(0, EC, c4, zeros)
        vtmp16[...] = l2acc
        pltpu.sync_copy(vtmp16, l2_h)

        def c5(a, carry):
            sl = pl.ds(a * 16, 16)
            vpx[sl] = vpx[sl] + vax[sl]
            vpy[sl] = vpy[sl] + vay[sl]
            vpz[sl] = vpz[sl] + vaz[sl]
            return carry
        lax.fori_loop(0, AC, c5, 0)

        pltpu.sync_copy(vpx, ox_h)
        pltpu.sync_copy(vpy, oy_h)
        pltpu.sync_copy(vpz, oz_h)
        pltpu.sync_copy(vrad, rad_h)


def _tc_body(xr_ref, yr_ref, zr_ref, rr_ref, xt_ref, yt_ref, zt_ref, rt_ref,
             l1_ref, l2_ref, ox_ref, oy_ref, oz_ref, loss_ref):
    b = pl.program_id(0)
    xi = xt_ref[...]
    yi = yt_ref[...]
    zi = zt_ref[...]
    xj = xr_ref[...]
    yj = yr_ref[...]
    zj = zr_ref[...]
    dx = xi - xj
    dy = yi - yj
    dz = zi - zj
    d2 = dx * dx + dy * dy + dz * dz
    rowid = b * 128 + lax.broadcasted_iota(_i32, (128, NP), 0)
    colid = lax.broadcasted_iota(_i32, (128, NP), 1)
    diag = rowid == colid
    valid = jnp.logical_not(diag) & (rowid < N) & (colid < N)
    dist = jnp.sqrt(jnp.where(diag, jnp.float32(1.0), d2))
    md = (rt_ref[...] + rr_ref[...]) * jnp.float32(0.8)
    gap = md - dist
    pen = jnp.where(valid, jnp.maximum(gap, jnp.float32(0.0)), jnp.float32(0.0))
    l3p = jnp.float32(0.5) * jnp.sum(pen * pen)
    cmask = valid & (dist < md) & (dist > jnp.float32(1e-8))
    w = jnp.where(cmask, gap * jnp.float32(0.0025) / dist, jnp.float32(0.0))
    s = jnp.sum(w, axis=1, keepdims=True)
    tx = jnp.sum(w * xj, axis=1, keepdims=True)
    ty = jnp.sum(w * yj, axis=1, keepdims=True)
    tz = jnp.sum(w * zj, axis=1, keepdims=True)
    ox_ref[...] = xi + (xi * s - tx)
    oy_ref[...] = yi + (yi * s - ty)
    oz_ref[...] = zi + (zi * s - tz)

    @pl.when(b == 0)
    def _():
        loss_ref[...] = jnp.reshape(
            jnp.sum(l1_ref[...]) + jnp.sum(l2_ref[...]) * jnp.float32(1.0 / 4000.0),
            (1, 1))

    loss_ref[...] = loss_ref[...] + jnp.reshape(l3p, (1, 1))

    @pl.when(b == NBLK - 1)
    def _():
        loss_ref[...] = loss_ref[...] * jnp.float32(0.1)


_tc_clash = pl.pallas_call(
    _tc_body,
    grid=(NBLK,),
    in_specs=[
        pl.BlockSpec((1, NP), lambda b: (0, 0)),
        pl.BlockSpec((1, NP), lambda b: (0, 0)),
        pl.BlockSpec((1, NP), lambda b: (0, 0)),
        pl.BlockSpec((1, NP), lambda b: (0, 0)),
        pl.BlockSpec((128, 1), lambda b: (b, 0)),
        pl.BlockSpec((128, 1), lambda b: (b, 0)),
        pl.BlockSpec((128, 1), lambda b: (b, 0)),
        pl.BlockSpec((128, 1), lambda b: (b, 0)),
        pl.BlockSpec((1, 16), lambda b: (0, 0)),
        pl.BlockSpec((1, 16), lambda b: (0, 0)),
    ],
    out_specs=[
        pl.BlockSpec((128, 1), lambda b: (b, 0)),
        pl.BlockSpec((128, 1), lambda b: (b, 0)),
        pl.BlockSpec((128, 1), lambda b: (b, 0)),
        pl.BlockSpec((1, 1), lambda b: (0, 0)),
    ],
    out_shape=[
        jax.ShapeDtypeStruct((NP, 1), _f32),
        jax.ShapeDtypeStruct((NP, 1), _f32),
        jax.ShapeDtypeStruct((NP, 1), _f32),
        jax.ShapeDtypeStruct((1, 1), _f32),
    ],
)


def kernel(pos, edge_index, atom_types):
    row = edge_index[0]
    col = edge_index[1]
    order = jnp.argsort(row).astype(_i32)
    px = jnp.zeros((NP,), _f32).at[:N].set(pos[:, 0])
    py = jnp.zeros((NP,), _f32).at[:N].set(pos[:, 1])
    pz = jnp.zeros((NP,), _f32).at[:N].set(pos[:, 2])
    atp = jnp.zeros((NP,), _i32).at[:N].set(atom_types)
    ox, oy, oz, rad, l1, l2 = _sc_forward(
        px, py, pz, row, col, order, atp,
        jnp.asarray(_MV_T), jnp.asarray(_LT_T), jnp.asarray(_VDW_T))
    fx, fy, fz, loss = _tc_clash(
        ox.reshape(1, NP), oy.reshape(1, NP), oz.reshape(1, NP),
        rad.reshape(1, NP),
        ox.reshape(NP, 1), oy.reshape(NP, 1), oz.reshape(NP, 1),
        rad.reshape(NP, 1),
        l1.reshape(1, 16), l2.reshape(1, 16))
    pos_new = jnp.concatenate([fx, fy, fz], axis=1)[:N]
    return pos_new, loss[0, 0]


# speedup vs baseline: 352.3877x; 1.2070x over previous
import functools

import numpy as np
import jax
import jax.numpy as jnp
from jax import lax
from jax.experimental import pallas as pl
from jax.experimental.pallas import tpu as pltpu
from jax.experimental.pallas import tpu_sc as plsc

N = 1000
NP = 1024
E = 4000
EC = E // 16
AC = NP // 16
NBLK = 8

_MV_T = np.full(16, 4.0, dtype=np.float32)
_MV_T[1] = 1.0; _MV_T[7] = 3.0; _MV_T[8] = 2.0; _MV_T[9] = 1.0
_LT = np.full((10, 10), 1.5, dtype=np.float32)
for (a, b), l in {(1, 6): 1.09, (6, 6): 1.54, (6, 7): 1.47, (6, 8): 1.43,
                  (6, 9): 1.35, (7, 7): 1.45, (7, 8): 1.40, (1, 7): 1.01,
                  (8, 8): 1.48, (1, 8): 0.96}.items():
    _LT[a, b] = l; _LT[b, a] = l
_LT_T = np.zeros(112, dtype=np.float32)
_LT_T[:100] = _LT.reshape(-1)
_VDW_T = np.full(16, 1.6, dtype=np.float32)
_VDW_T[1] = 1.2; _VDW_T[6] = 1.7; _VDW_T[7] = 1.55; _VDW_T[8] = 1.52; _VDW_T[9] = 1.47

_f32 = jnp.float32
_i32 = jnp.int32


def _rsqrt(x):
    xc = jnp.maximum(x, jnp.float32(1e-35))
    i = lax.bitcast_convert_type(xc, jnp.int32)
    i = jnp.int32(0x5F375A86) - lax.shift_right_logical(i, 1)
    y = lax.bitcast_convert_type(i, jnp.float32)
    for _ in range(3):
        y = y * (jnp.float32(1.5) - jnp.float32(0.5) * xc * y * y)
    return y


def _rsqrt2(x):
    xc = jnp.maximum(x, jnp.float32(1e-35))
    i = lax.bitcast_convert_type(xc, jnp.int32)
    i = jnp.int32(0x5F375A86) - lax.shift_right_logical(i, 1)
    y = lax.bitcast_convert_type(i, jnp.float32)
    for _ in range(2):
        y = y * (jnp.float32(1.5) - jnp.float32(0.5) * xc * y * y)
    return y


_sc_mesh = plsc.VectorSubcoreMesh(core_axis_name="c", subcore_axis_name="s")


@functools.partial(
    pl.kernel,
    out_type=(
        jax.ShapeDtypeStruct((NP,), _f32),
        jax.ShapeDtypeStruct((NP,), _f32),
        jax.ShapeDtypeStruct((NP,), _f32),
        jax.ShapeDtypeStruct((NP,), _f32),
        jax.ShapeDtypeStruct((16,), _f32),
        jax.ShapeDtypeStruct((16,), _f32),
    ),
    mesh=_sc_mesh,
    compiler_params=pltpu.CompilerParams(needs_layout_passes=False),
    scratch_types=[
        pltpu.VMEM((NP * 4,), _f32),
        pltpu.VMEM((NP,), _f32),
        pltpu.VMEM((NP,), _f32),
        pltpu.VMEM((NP,), _f32),
        pltpu.VMEM((E,), _i32),
        pltpu.VMEM((E,), _i32),
        pltpu.VMEM((E,), _i32),
        pltpu.VMEM((E + 16,), _i32),
        pltpu.VMEM((E + 16,), _i32),
        pltpu.VMEM((E + 16,), _f32),
        pltpu.VMEM((NP,), _i32),
        pltpu.VMEM((NP,), _f32),
        pltpu.VMEM((NP,), _f32),
        pltpu.VMEM((NP,), _f32),
        pltpu.VMEM((NP,), _f32),
        pltpu.VMEM((NP,), _f32),
        pltpu.VMEM((NP,), _f32),
        pltpu.VMEM((16,), _f32),
        pltpu.VMEM((112,), _f32),
        pltpu.VMEM((16,), _f32),
        pltpu.VMEM((16,), _f32),
    ],
)
def _sc_forward(pk_h, row_h, col_h, ord_h, at_h, mv_h, lt_h, vdw_h,
                ox_h, oy_h, oz_h, rad_h, l1_h, l2_h,
                vpk, vpx, vpy, vpz, vrow, vcol, vord, vrs, vcs, vef, vat, vcnt, vfac,
                vax, vay, vaz, vrad, vmv, vlt, vvdw, vtmp16):
    is_w0 = (lax.axis_index("c") == 0) & (lax.axis_index("s") == 0)

    @pl.when(is_w0)
    def _():
        pltpu.sync_copy(pk_h, vpk)
        pltpu.sync_copy(row_h, vrow)
        pltpu.sync_copy(col_h, vcol)
        pltpu.sync_copy(ord_h, vord)
        pltpu.sync_copy(at_h, vat)
        pltpu.sync_copy(mv_h, vmv)
        pltpu.sync_copy(lt_h, vlt)
        pltpu.sync_copy(vdw_h, vvdw)

        iota = lax.iota(_i32, 16)
        m0 = iota == 0
        mall = iota < 16
        zeros = jnp.zeros((16,), _f32)
        ones = jnp.ones((16,), _f32)

        def zinit(a, carry):
            sl = pl.ds(a * 16, 16)
            vcnt[sl] = zeros
            vax[sl] = zeros
            vay[sl] = zeros
            vaz[sl] = zeros
            return carry
        lax.fori_loop(0, AC, zinit, 0)

        def c1(k, carry):
            sl = pl.ds(k * 16, 16)
            plsc.addupdate_scatter(vcnt, [vrow[sl]], ones, mask=mall)
            return carry
        lax.fori_loop(0, EC, c1, 0)

        def c2(a, l1acc):
            sl = pl.ds(a * 16, 16)
            atv = vat[sl]
            mvv = plsc.load_gather(vmv, [atv])
            cv = vcnt[sl]
            viol = cv - mvv
            m = cv > mvv
            l1acc = l1acc + jnp.where(m, viol * viol, jnp.float32(0.0))
            vfac[sl] = jnp.where(m, viol * jnp.float32(0.001), jnp.float32(0.0))
            vrad[sl] = plsc.load_gather(vvdw, [atv])
            return l1acc
        l1acc = lax.fori_loop(0, AC, c2, zeros)
        vtmp16[...] = l1acc
        pltpu.sync_copy(vtmp16, l1_h)

        def c3(k, off):
            sl = pl.ds(k * 16, 16)
            ov = vord[sl]
            rv = plsc.load_gather(vrow, [ov])
            cv = plsc.load_gather(vcol, [ov])
            fv = plsc.load_gather(vfac, [rv])
            m = fv > jnp.float32(0.0)
            plsc.store_compressed(vrs.at[pl.ds(off, 16)], rv * 4, mask=m)
            plsc.store_compressed(vcs.at[pl.ds(off, 16)], cv * 4, mask=m)
            plsc.store_compressed(vef.at[pl.ds(off, 16)], fv, mask=m)
            return off + jnp.max(plsc.all_reduce_population_count(m))
        nkeep = lax.fori_loop(0, EC, c3, jnp.int32(0))
        vrs[pl.ds(nkeep, 16)] = jnp.full((16,), 4 * (NP - 1), _i32)
        vcs[pl.ds(nkeep, 16)] = jnp.full((16,), 4 * (NP - 1), _i32)
        vef[pl.ds(nkeep, 16)] = zeros

        m3 = iota < 3
        off3 = jnp.where(m3, iota, 0)
        r1 = jnp.where(m3, lax.rem(iota + 1, jnp.int32(3)), iota)
        r2 = jnp.where(m3, lax.rem(iota + 2, jnp.int32(3)), iota)
        nch = lax.shift_right_logical(nkeep + jnp.int32(15), 4)

        def seqc(c, carry):
            sl = pl.ds(c * 16, 16)
            rv4 = vrs[sl]
            cv4 = vcs[sl]
            fvv = vef[sl]
            for kk in range(16):
                lk = jnp.full((16,), kk, _i32)
                ib = rv4.at[lk].get(mode="promise_in_bounds")
                jb = cv4.at[lk].get(mode="promise_in_bounds")
                fb = fvv.at[lk].get(mode="promise_in_bounds")
                idxi = ib + off3
                idxj = jb + off3
                pi = plsc.load_gather(vpk, [idxi], mask=m3)
                pj = plsc.load_gather(vpk, [idxj], mask=m3)
                d = pi - pj
                t = d * d
                d2 = (t + t.at[r1].get(mode="promise_in_bounds")
                      + t.at[r2].get(mode="promise_in_bounds"))
                f = fb * _rsqrt2(d2)
                plsc.store_scatter(vpk, [idxi], pi + d * f, mask=m3)
            return carry
        lax.fori_loop(0, nch, seqc, 0)

        def c4(k, l2acc):
            sl = pl.ds(k * 16, 16)
            rv = vrow[sl]
            cv = vcol[sl]
            r4 = rv * 4
            c4_ = cv * 4
            xi = plsc.load_gather(vpk, [r4])
            yi = plsc.load_gather(vpk, [r4 + 1])
            zi = plsc.load_gather(vpk, [r4 + 2])
            xj = plsc.load_gather(vpk, [c4_])
            yj = plsc.load_gather(vpk, [c4_ + 1])
            zj = plsc.load_gather(vpk, [c4_ + 2])
            bx = xi - xj
            by = yi - yj
            bz = zi - zj
            d2 = bx * bx + by * by + bz * bz
            cur = d2 * _rsqrt(d2)
            ati = plsc.load_gather(vat, [rv])
            atj = plsc.load_gather(vat, [cv])
            tv = plsc.load_gather(vlt, [ati * 10 + atj])
            le = cur - tv
            l2acc = l2acc + le * le
            rc = _rsqrt(cur + jnp.float32(1e-8))
            inv = rc * rc
            ratio = jnp.clip(tv * inv, jnp.float32(0.98), jnp.float32(1.02))
            g = (ratio - jnp.float32(1.0)) * jnp.float32(0.005)
            plsc.addupdate_scatter(vax, [rv], bx * g, mask=mall)
            plsc.addupdate_scatter(vay, [rv], by * g, mask=mall)
            plsc.addupdate_scatter(vaz, [rv], bz * g, mask=mall)
            plsc.addupdate_scatter(vax, [cv], -(bx * g), mask=mall)
            plsc.addupdate_scatter(vay, [cv], -(by * g), mask=mall)
            plsc.addupdate_scatter(vaz, [cv], -(bz * g), mask=mall)
            return l2acc
        l2acc = lax.fori_loop(0, EC, c4, zeros)
        vtmp16[...] = l2acc
        pltpu.sync_copy(vtmp16, l2_h)

        iota4 = iota * 4
        def c5(a, carry):
            sl = pl.ds(a * 16, 16)
            base = jnp.full((16,), a * 64, _i32) + iota4
            vpx[sl] = plsc.load_gather(vpk, [base]) + vax[sl]
            vpy[sl] = plsc.load_gather(vpk, [base + 1]) + vay[sl]
            vpz[sl] = plsc.load_gather(vpk, [base + 2]) + vaz[sl]
            return carry
        lax.fori_loop(0, AC, c5, 0)

        pltpu.sync_copy(vpx, ox_h)
        pltpu.sync_copy(vpy, oy_h)
        pltpu.sync_copy(vpz, oz_h)
        pltpu.sync_copy(vrad, rad_h)


def _tc_body(xr_ref, yr_ref, zr_ref, rr_ref, xt_ref, yt_ref, zt_ref, rt_ref,
             l1_ref, l2_ref, ox_ref, oy_ref, oz_ref, loss_ref):
    b = pl.program_id(0)
    xi = xt_ref[...]
    yi = yt_ref[...]
    zi = zt_ref[...]
    xj = xr_ref[...]
    yj = yr_ref[...]
    zj = zr_ref[...]
    dx = xi - xj
    dy = yi - yj
    dz = zi - zj
    d2 = dx * dx + dy * dy + dz * dz
    rowid = b * 128 + lax.broadcasted_iota(_i32, (128, NP), 0)
    colid = lax.broadcasted_iota(_i32, (128, NP), 1)
    diag = rowid == colid
    valid = jnp.logical_not(diag) & (rowid < N) & (colid < N)
    dist = jnp.sqrt(jnp.where(diag, jnp.float32(1.0), d2))
    md = (rt_ref[...] + rr_ref[...]) * jnp.float32(0.8)
    gap = md - dist
    pen = jnp.where(valid, jnp.maximum(gap, jnp.float32(0.0)), jnp.float32(0.0))
    l3p = jnp.float32(0.5) * jnp.sum(pen * pen)
    cmask = valid & (dist < md) & (dist > jnp.float32(1e-8))
    w = jnp.where(cmask, gap * jnp.float32(0.0025) / dist, jnp.float32(0.0))
    s = jnp.sum(w, axis=1, keepdims=True)
    tx = jnp.sum(w * xj, axis=1, keepdims=True)
    ty = jnp.sum(w * yj, axis=1, keepdims=True)
    tz = jnp.sum(w * zj, axis=1, keepdims=True)
    ox_ref[...] = xi + (xi * s - tx)
    oy_ref[...] = yi + (yi * s - ty)
    oz_ref[...] = zi + (zi * s - tz)

    @pl.when(b == 0)
    def _():
        loss_ref[...] = jnp.reshape(
            jnp.sum(l1_ref[...]) + jnp.sum(l2_ref[...]) * jnp.float32(1.0 / 4000.0),
            (1, 1))

    loss_ref[...] = loss_ref[...] + jnp.reshape(l3p, (1, 1))

    @pl.when(b == NBLK - 1)
    def _():
        loss_ref[...] = loss_ref[...] * jnp.float32(0.1)


_tc_clash = pl.pallas_call(
    _tc_body,
    grid=(NBLK,),
    in_specs=[
        pl.BlockSpec((1, NP), lambda b: (0, 0)),
        pl.BlockSpec((1, NP), lambda b: (0, 0)),
        pl.BlockSpec((1, NP), lambda b: (0, 0)),
        pl.BlockSpec((1, NP), lambda b: (0, 0)),
        pl.BlockSpec((128, 1), lambda b: (b, 0)),
        pl.BlockSpec((128, 1), lambda b: (b, 0)),
        pl.BlockSpec((128, 1), lambda b: (b, 0)),
        pl.BlockSpec((128, 1), lambda b: (b, 0)),
        pl.BlockSpec((1, 16), lambda b: (0, 0)),
        pl.BlockSpec((1, 16), lambda b: (0, 0)),
    ],
    out_specs=[
        pl.BlockSpec((128, 1), lambda b: (b, 0)),
        pl.BlockSpec((128, 1), lambda b: (b, 0)),
        pl.BlockSpec((128, 1), lambda b: (b, 0)),
        pl.BlockSpec((1, 1), lambda b: (0, 0)),
    ],
    out_shape=[
        jax.ShapeDtypeStruct((NP, 1), _f32),
        jax.ShapeDtypeStruct((NP, 1), _f32),
        jax.ShapeDtypeStruct((NP, 1), _f32),
        jax.ShapeDtypeStruct((1, 1), _f32),
    ],
)


def kernel(pos, edge_index, atom_types):
    row = edge_index[0]
    col = edge_index[1]
    order = jnp.argsort(row).astype(_i32)
    pk = jnp.zeros((NP, 4), _f32).at[:N, :3].set(pos).reshape(-1)
    atp = jnp.zeros((NP,), _i32).at[:N].set(atom_types)
    ox, oy, oz, rad, l1, l2 = _sc_forward(
        pk, row, col, order, atp,
        jnp.asarray(_MV_T), jnp.asarray(_LT_T), jnp.asarray(_VDW_T))
    fx, fy, fz, loss = _tc_clash(
        ox.reshape(1, NP), oy.reshape(1, NP), oz.reshape(1, NP),
        rad.reshape(1, NP),
        ox.reshape(NP, 1), oy.reshape(NP, 1), oz.reshape(NP, 1),
        rad.reshape(NP, 1),
        l1.reshape(1, 16), l2.reshape(1, 16))
    pos_new = jnp.concatenate([fx, fy, fz], axis=1)[:N]
    return pos_new, loss[0, 0]


# speedup vs baseline: 389.2336x; 1.1046x over previous
import functools

import numpy as np
import jax
import jax.numpy as jnp
from jax import lax
from jax.experimental import pallas as pl
from jax.experimental.pallas import tpu as pltpu
from jax.experimental.pallas import tpu_sc as plsc

N = 1000
NP = 1024
E = 4000
EC = E // 16
AC = NP // 16
NBLK = 8

_MV_T = np.full(16, 4.0, dtype=np.float32)
_MV_T[1] = 1.0; _MV_T[7] = 3.0; _MV_T[8] = 2.0; _MV_T[9] = 1.0
_LT = np.full((10, 10), 1.5, dtype=np.float32)
for (a, b), l in {(1, 6): 1.09, (6, 6): 1.54, (6, 7): 1.47, (6, 8): 1.43,
                  (6, 9): 1.35, (7, 7): 1.45, (7, 8): 1.40, (1, 7): 1.01,
                  (8, 8): 1.48, (1, 8): 0.96}.items():
    _LT[a, b] = l; _LT[b, a] = l
_LT_T = np.zeros(112, dtype=np.float32)
_LT_T[:100] = _LT.reshape(-1)
_VDW_T = np.full(16, 1.6, dtype=np.float32)
_VDW_T[1] = 1.2; _VDW_T[6] = 1.7; _VDW_T[7] = 1.55; _VDW_T[8] = 1.52; _VDW_T[9] = 1.47

_f32 = jnp.float32
_i32 = jnp.int32


def _rsqrt(x):
    xc = jnp.maximum(x, jnp.float32(1e-35))
    i = lax.bitcast_convert_type(xc, jnp.int32)
    i = jnp.int32(0x5F375A86) - lax.shift_right_logical(i, 1)
    y = lax.bitcast_convert_type(i, jnp.float32)
    for _ in range(3):
        y = y * (jnp.float32(1.5) - jnp.float32(0.5) * xc * y * y)
    return y


def _rsqrt2(x):
    xc = jnp.maximum(x, jnp.float32(1e-35))
    i = lax.bitcast_convert_type(xc, jnp.int32)
    i = jnp.int32(0x5F375A86) - lax.shift_right_logical(i, 1)
    y = lax.bitcast_convert_type(i, jnp.float32)
    y = y * (jnp.float32(1.5) - jnp.float32(0.5) * xc * y * y)
    return y


_sc_mesh = plsc.VectorSubcoreMesh(core_axis_name="c", subcore_axis_name="s")


@functools.partial(
    pl.kernel,
    out_type=(
        jax.ShapeDtypeStruct((NP,), _f32),
        jax.ShapeDtypeStruct((NP,), _f32),
        jax.ShapeDtypeStruct((NP,), _f32),
        jax.ShapeDtypeStruct((NP,), _f32),
        jax.ShapeDtypeStruct((16,), _f32),
        jax.ShapeDtypeStruct((16,), _f32),
    ),
    mesh=_sc_mesh,
    compiler_params=pltpu.CompilerParams(needs_layout_passes=False),
    scratch_types=[
        pltpu.VMEM((NP * 4,), _f32),
        pltpu.VMEM((NP,), _f32),
        pltpu.VMEM((NP,), _f32),
        pltpu.VMEM((NP,), _f32),
        pltpu.VMEM((E,), _i32),
        pltpu.VMEM((E,), _i32),
        pltpu.VMEM((E,), _i32),
        pltpu.VMEM((E + 16,), _i32),
        pltpu.VMEM((E + 16,), _i32),
        pltpu.VMEM((E + 16,), _f32),
        pltpu.VMEM((NP,), _i32),
        pltpu.VMEM((NP,), _f32),
        pltpu.VMEM((NP,), _f32),
        pltpu.VMEM((NP,), _f32),
        pltpu.VMEM((NP,), _f32),
        pltpu.VMEM((NP,), _f32),
        pltpu.VMEM((NP,), _f32),
        pltpu.VMEM((16,), _f32),
        pltpu.VMEM((112,), _f32),
        pltpu.VMEM((16,), _f32),
        pltpu.VMEM((16,), _f32),
    ],
)
def _sc_forward(pk_h, row_h, col_h, ord_h, at_h, mv_h, lt_h, vdw_h,
                ox_h, oy_h, oz_h, rad_h, l1_h, l2_h,
                vpk, vpx, vpy, vpz, vrow, vcol, vord, vrs, vcs, vef, vat, vcnt, vfac,
                vax, vay, vaz, vrad, vmv, vlt, vvdw, vtmp16):
    is_w0 = (lax.axis_index("c") == 0) & (lax.axis_index("s") == 0)

    @pl.when(is_w0)
    def _():
        pltpu.sync_copy(pk_h, vpk)
        pltpu.sync_copy(row_h, vrow)
        pltpu.sync_copy(col_h, vcol)
        pltpu.sync_copy(ord_h, vord)
        pltpu.sync_copy(at_h, vat)
        pltpu.sync_copy(mv_h, vmv)
        pltpu.sync_copy(lt_h, vlt)
        pltpu.sync_copy(vdw_h, vvdw)

        iota = lax.iota(_i32, 16)
        m0 = iota == 0
        mall = iota < 16
        zeros = jnp.zeros((16,), _f32)
        ones = jnp.ones((16,), _f32)

        def zinit(a, carry):
            sl = pl.ds(a * 16, 16)
            vcnt[sl] = zeros
            vax[sl] = zeros
            vay[sl] = zeros
            vaz[sl] = zeros
            return carry
        lax.fori_loop(0, AC, zinit, 0)

        def c1(k, carry):
            sl = pl.ds(k * 16, 16)
            plsc.addupdate_scatter(vcnt, [vrow[sl]], ones, mask=mall)
            return carry
        lax.fori_loop(0, EC, c1, 0)

        def c2(a, l1acc):
            sl = pl.ds(a * 16, 16)
            atv = vat[sl]
            mvv = plsc.load_gather(vmv, [atv])
            cv = vcnt[sl]
            viol = cv - mvv
            m = cv > mvv
            l1acc = l1acc + jnp.where(m, viol * viol, jnp.float32(0.0))
            vfac[sl] = jnp.where(m, viol * jnp.float32(0.001), jnp.float32(0.0))
            vrad[sl] = plsc.load_gather(vvdw, [atv])
            return l1acc
        l1acc = lax.fori_loop(0, AC, c2, zeros)
        vtmp16[...] = l1acc
        pltpu.sync_copy(vtmp16, l1_h)

        def c3(k, off):
            sl = pl.ds(k * 16, 16)
            ov = vord[sl]
            rv = plsc.load_gather(vrow, [ov])
            cv = plsc.load_gather(vcol, [ov])
            fv = plsc.load_gather(vfac, [rv])
            m = fv > jnp.float32(0.0)
            plsc.store_compressed(vrs.at[pl.ds(off, 16)], rv * 4, mask=m)
            plsc.store_compressed(vcs.at[pl.ds(off, 16)], cv * 4, mask=m)
            plsc.store_compressed(vef.at[pl.ds(off, 16)], fv, mask=m)
            return off + jnp.max(plsc.all_reduce_population_count(m))
        nkeep = lax.fori_loop(0, EC, c3, jnp.int32(0))
        vrs[pl.ds(nkeep, 16)] = jnp.full((16,), 4 * (NP - 1), _i32)
        vcs[pl.ds(nkeep, 16)] = jnp.full((16,), 4 * (NP - 1), _i32)
        vef[pl.ds(nkeep, 16)] = zeros

        m3 = iota < 3
        off3 = jnp.where(m3, iota, 0)
        r1 = jnp.where(m3, lax.rem(iota + 1, jnp.int32(3)), iota)
        r2 = jnp.where(m3, lax.rem(iota + 2, jnp.int32(3)), iota)
        nch = lax.shift_right_logical(nkeep + jnp.int32(15), 4)

        def seqc(c, carry):
            sl = pl.ds(c * 16, 16)
            rv4 = vrs[sl]
            cv4 = vcs[sl]
            fvv = vef[sl]
            for kk in range(16):
                lk = jnp.full((16,), kk, _i32)
                ib = rv4.at[lk].get(mode="promise_in_bounds")
                jb = cv4.at[lk].get(mode="promise_in_bounds")
                fb = fvv.at[lk].get(mode="promise_in_bounds")
                idxi = ib + off3
                idxj = jb + off3
                pi = plsc.load_gather(vpk, [idxi], mask=m3)
                pj = plsc.load_gather(vpk, [idxj], mask=m3)
                d = pi - pj
                t = d * d
                d2 = (t + t.at[r1].get(mode="promise_in_bounds")
                      + t.at[r2].get(mode="promise_in_bounds"))
                f = fb * _rsqrt2(d2)
                plsc.store_scatter(vpk, [idxi], pi + d * f, mask=m3)
            return carry
        lax.fori_loop(0, nch, seqc, 0)

        def c4(k, l2acc):
            sl = pl.ds(k * 16, 16)
            rv = vrow[sl]
            cv = vcol[sl]
            r4 = rv * 4
            c4_ = cv * 4
            xi = plsc.load_gather(vpk, [r4])
            yi = plsc.load_gather(vpk, [r4 + 1])
            zi = plsc.load_gather(vpk, [r4 + 2])
            xj = plsc.load_gather(vpk, [c4_])
            yj = plsc.load_gather(vpk, [c4_ + 1])
            zj = plsc.load_gather(vpk, [c4_ + 2])
            bx = xi - xj
            by = yi - yj
            bz = zi - zj
            d2 = bx * bx + by * by + bz * bz
            cur = d2 * _rsqrt(d2)
            ati = plsc.load_gather(vat, [rv])
            atj = plsc.load_gather(vat, [cv])
            tv = plsc.load_gather(vlt, [ati * 10 + atj])
            le = cur - tv
            l2acc = l2acc + le * le
            rc = _rsqrt(cur + jnp.float32(1e-8))
            inv = rc * rc
            ratio = jnp.clip(tv * inv, jnp.float32(0.98), jnp.float32(1.02))
            g = (ratio - jnp.float32(1.0)) * jnp.float32(0.005)
            plsc.addupdate_scatter(vax, [rv], bx * g, mask=mall)
            plsc.addupdate_scatter(vay, [rv], by * g, mask=mall)
            plsc.addupdate_scatter(vaz, [rv], bz * g, mask=mall)
            plsc.addupdate_scatter(vax, [cv], -(bx * g), mask=mall)
            plsc.addupdate_scatter(vay, [cv], -(by * g), mask=mall)
            plsc.addupdate_scatter(vaz, [cv], -(bz * g), mask=mall)
            return l2acc
        l2acc = lax.fori_loop(0, EC, c4, zeros)
        vtmp16[...] = l2acc
        pltpu.sync_copy(vtmp16, l2_h)

        iota4 = iota * 4
        def c5(a, carry):
            sl = pl.ds(a * 16, 16)
            base = jnp.full((16,), a * 64, _i32) + iota4
            vpx[sl] = plsc.load_gather(vpk, [base]) + vax[sl]
            vpy[sl] = plsc.load_gather(vpk, [base + 1]) + vay[sl]
            vpz[sl] = plsc.load_gather(vpk, [base + 2]) + vaz[sl]
            return carry
        lax.fori_loop(0, AC, c5, 0)

        pltpu.sync_copy(vpx, ox_h)
        pltpu.sync_copy(vpy, oy_h)
        pltpu.sync_copy(vpz, oz_h)
        pltpu.sync_copy(vrad, rad_h)


def _tc_body(xr_ref, yr_ref, zr_ref, rr_ref, xt_ref, yt_ref, zt_ref, rt_ref,
             l1_ref, l2_ref, ox_ref, oy_ref, oz_ref, loss_ref):
    b = pl.program_id(0)
    xi = xt_ref[...]
    yi = yt_ref[...]
    zi = zt_ref[...]
    xj = xr_ref[...]
    yj = yr_ref[...]
    zj = zr_ref[...]
    dx = xi - xj
    dy = yi - yj
    dz = zi - zj
    d2 = dx * dx + dy * dy + dz * dz
    rowid = b * 128 + lax.broadcasted_iota(_i32, (128, NP), 0)
    colid = lax.broadcasted_iota(_i32, (128, NP), 1)
    diag = rowid == colid
    valid = jnp.logical_not(diag) & (rowid < N) & (colid < N)
    dist = jnp.sqrt(jnp.where(diag, jnp.float32(1.0), d2))
    md = (rt_ref[...] + rr_ref[...]) * jnp.float32(0.8)
    gap = md - dist
    pen = jnp.where(valid, jnp.maximum(gap, jnp.float32(0.0)), jnp.float32(0.0))
    l3p = jnp.float32(0.5) * jnp.sum(pen * pen)
    cmask = valid & (dist < md) & (dist > jnp.float32(1e-8))
    w = jnp.where(cmask, gap * jnp.float32(0.0025) / dist, jnp.float32(0.0))
    s = jnp.sum(w, axis=1, keepdims=True)
    tx = jnp.sum(w * xj, axis=1, keepdims=True)
    ty = jnp.sum(w * yj, axis=1, keepdims=True)
    tz = jnp.sum(w * zj, axis=1, keepdims=True)
    ox_ref[...] = xi + (xi * s - tx)
    oy_ref[...] = yi + (yi * s - ty)
    oz_ref[...] = zi + (zi * s - tz)

    @pl.when(b == 0)
    def _():
        loss_ref[...] = jnp.reshape(
            jnp.sum(l1_ref[...]) + jnp.sum(l2_ref[...]) * jnp.float32(1.0 / 4000.0),
            (1, 1))

    loss_ref[...] = loss_ref[...] + jnp.reshape(l3p, (1, 1))

    @pl.when(b == NBLK - 1)
    def _():
        loss_ref[...] = loss_ref[...] * jnp.float32(0.1)


_tc_clash = pl.pallas_call(
    _tc_body,
    grid=(NBLK,),
    in_specs=[
        pl.BlockSpec((1, NP), lambda b: (0, 0)),
        pl.BlockSpec((1, NP), lambda b: (0, 0)),
        pl.BlockSpec((1, NP), lambda b: (0, 0)),
        pl.BlockSpec((1, NP), lambda b: (0, 0)),
        pl.BlockSpec((128, 1), lambda b: (b, 0)),
        pl.BlockSpec((128, 1), lambda b: (b, 0)),
        pl.BlockSpec((128, 1), lambda b: (b, 0)),
        pl.BlockSpec((128, 1), lambda b: (b, 0)),
        pl.BlockSpec((1, 16), lambda b: (0, 0)),
        pl.BlockSpec((1, 16), lambda b: (0, 0)),
    ],
    out_specs=[
        pl.BlockSpec((128, 1), lambda b: (b, 0)),
        pl.BlockSpec((128, 1), lambda b: (b, 0)),
        pl.BlockSpec((128, 1), lambda b: (b, 0)),
        pl.BlockSpec((1, 1), lambda b: (0, 0)),
    ],
    out_shape=[
        jax.ShapeDtypeStruct((NP, 1), _f32),
        jax.ShapeDtypeStruct((NP, 1), _f32),
        jax.ShapeDtypeStruct((NP, 1), _f32),
        jax.ShapeDtypeStruct((1, 1), _f32),
    ],
)


def kernel(pos, edge_index, atom_types):
    row = edge_index[0]
    col = edge_index[1]
    order = jnp.argsort(row).astype(_i32)
    pk = jnp.zeros((NP, 4), _f32).at[:N, :3].set(pos).reshape(-1)
    atp = jnp.zeros((NP,), _i32).at[:N].set(atom_types)
    ox, oy, oz, rad, l1, l2 = _sc_forward(
        pk, row, col, order, atp,
        jnp.asarray(_MV_T), jnp.asarray(_LT_T), jnp.asarray(_VDW_T))
    fx, fy, fz, loss = _tc_clash(
        ox.reshape(1, NP), oy.reshape(1, NP), oz.reshape(1, NP),
        rad.reshape(1, NP),
        ox.reshape(NP, 1), oy.reshape(NP, 1), oz.reshape(NP, 1),
        rad.reshape(NP, 1),
        l1.reshape(1, 16), l2.reshape(1, 16))
    pos_new = jnp.concatenate([fx, fy, fz], axis=1)[:N]
    return pos_new, loss[0, 0]


# speedup vs baseline: 405.8597x; 1.0427x over previous
import functools

import numpy as np
import jax
import jax.numpy as jnp
from jax import lax
from jax.experimental import pallas as pl
from jax.experimental.pallas import tpu as pltpu
from jax.experimental.pallas import tpu_sc as plsc

N = 1000
NP = 1024
E = 4000
EC = E // 16
AC = NP // 16
NBLK = 8

_MV_T = np.full(16, 4.0, dtype=np.float32)
_MV_T[1] = 1.0; _MV_T[7] = 3.0; _MV_T[8] = 2.0; _MV_T[9] = 1.0
_LT = np.full((10, 10), 1.5, dtype=np.float32)
for (a, b), l in {(1, 6): 1.09, (6, 6): 1.54, (6, 7): 1.47, (6, 8): 1.43,
                  (6, 9): 1.35, (7, 7): 1.45, (7, 8): 1.40, (1, 7): 1.01,
                  (8, 8): 1.48, (1, 8): 0.96}.items():
    _LT[a, b] = l; _LT[b, a] = l
_LT_T = np.zeros(112, dtype=np.float32)
_LT_T[:100] = _LT.reshape(-1)
_VDW_T = np.full(16, 1.6, dtype=np.float32)
_VDW_T[1] = 1.2; _VDW_T[6] = 1.7; _VDW_T[7] = 1.55; _VDW_T[8] = 1.52; _VDW_T[9] = 1.47

_f32 = jnp.float32
_i32 = jnp.int32


def _rsqrt_n(x, iters):
    xc = jnp.maximum(x, jnp.float32(1e-35))
    i = lax.bitcast_convert_type(xc, jnp.int32)
    i = jnp.int32(0x5F375A86) - lax.shift_right_logical(i, 1)
    y = lax.bitcast_convert_type(i, jnp.float32)
    for _ in range(iters):
        y = y * (jnp.float32(1.5) - jnp.float32(0.5) * xc * y * y)
    return y


def _rsqrt(x):
    return _rsqrt_n(x, 3)


def _rsqrt2(x):
    return _rsqrt_n(x, 1)


_sc_mesh = plsc.VectorSubcoreMesh(core_axis_name="c", subcore_axis_name="s")


@functools.partial(
    pl.kernel,
    out_type=(
        jax.ShapeDtypeStruct((NP,), _f32),
        jax.ShapeDtypeStruct((NP,), _f32),
        jax.ShapeDtypeStruct((NP,), _f32),
        jax.ShapeDtypeStruct((NP,), _f32),
        jax.ShapeDtypeStruct((16,), _f32),
        jax.ShapeDtypeStruct((16,), _f32),
    ),
    mesh=_sc_mesh,
    compiler_params=pltpu.CompilerParams(needs_layout_passes=False),
    scratch_types=[
        pltpu.VMEM((NP * 4,), _f32),
        pltpu.VMEM((NP,), _f32),
        pltpu.VMEM((NP,), _f32),
        pltpu.VMEM((NP,), _f32),
        pltpu.VMEM((E,), _i32),
        pltpu.VMEM((E,), _i32),
        pltpu.VMEM((E,), _i32),
        pltpu.VMEM((E + 16,), _i32),
        pltpu.VMEM((E + 16,), _i32),
        pltpu.VMEM((E + 16,), _f32),
        pltpu.VMEM((NP,), _i32),
        pltpu.VMEM((NP,), _f32),
        pltpu.VMEM((NP,), _f32),
        pltpu.VMEM((NP,), _f32),
        pltpu.VMEM((NP,), _f32),
        pltpu.VMEM((NP,), _f32),
        pltpu.VMEM((NP,), _f32),
        pltpu.VMEM((16,), _f32),
        pltpu.VMEM((112,), _f32),
        pltpu.VMEM((16,), _f32),
        pltpu.VMEM((16,), _f32),
    ],
)
def _sc_forward(pk_h, row_h, col_h, ord_h, at_h, mv_h, lt_h, vdw_h,
                ox_h, oy_h, oz_h, rad_h, l1_h, l2_h,
                vpk, vpx, vpy, vpz, vrow, vcol, vord, vrs, vcs, vef, vat, vcnt, vfac,
                vax, vay, vaz, vrad, vmv, vlt, vvdw, vtmp16):
    is_w0 = (lax.axis_index("c") == 0) & (lax.axis_index("s") == 0)

    @pl.when(is_w0)
    def _():
        pltpu.sync_copy(pk_h, vpk)
        pltpu.sync_copy(row_h, vrow)
        pltpu.sync_copy(col_h, vcol)
        pltpu.sync_copy(ord_h, vord)
        pltpu.sync_copy(at_h, vat)
        pltpu.sync_copy(mv_h, vmv)
        pltpu.sync_copy(lt_h, vlt)
        pltpu.sync_copy(vdw_h, vvdw)

        iota = lax.iota(_i32, 16)
        m0 = iota == 0
        mall = iota < 16
        zeros = jnp.zeros((16,), _f32)
        ones = jnp.ones((16,), _f32)

        def zinit(a, carry):
            sl = pl.ds(a * 16, 16)
            vcnt[sl] = zeros
            vax[sl] = zeros
            vay[sl] = zeros
            vaz[sl] = zeros
            return carry
        lax.fori_loop(0, AC, zinit, 0)

        def c1(k, carry):
            sl = pl.ds(k * 16, 16)
            plsc.addupdate_scatter(vcnt, [vrow[sl]], ones, mask=mall)
            return carry
        lax.fori_loop(0, EC, c1, 0)

        def c2(a, l1acc):
            sl = pl.ds(a * 16, 16)
            atv = vat[sl]
            mvv = plsc.load_gather(vmv, [atv])
            cv = vcnt[sl]
            viol = cv - mvv
            m = cv > mvv
            l1acc = l1acc + jnp.where(m, viol * viol, jnp.float32(0.0))
            vfac[sl] = jnp.where(m, viol * jnp.float32(0.001), jnp.float32(0.0))
            vrad[sl] = plsc.load_gather(vvdw, [atv])
            return l1acc
        l1acc = lax.fori_loop(0, AC, c2, zeros)
        vtmp16[...] = l1acc
        pltpu.sync_copy(vtmp16, l1_h)

        def c3(k, off):
            sl = pl.ds(k * 16, 16)
            ov = vord[sl]
            rv = plsc.load_gather(vrow, [ov])
            cv = plsc.load_gather(vcol, [ov])
            fv = plsc.load_gather(vfac, [rv])
            m = fv > jnp.float32(0.0)
            plsc.store_compressed(vrs.at[pl.ds(off, 16)], rv * 4, mask=m)
            plsc.store_compressed(vcs.at[pl.ds(off, 16)], cv * 4, mask=m)
            plsc.store_compressed(vef.at[pl.ds(off, 16)], fv, mask=m)
            return off + jnp.max(plsc.all_reduce_population_count(m))
        nkeep = lax.fori_loop(0, EC, c3, jnp.int32(0))
        vrs[pl.ds(nkeep, 16)] = jnp.full((16,), 4 * (NP - 1), _i32)
        vcs[pl.ds(nkeep, 16)] = jnp.full((16,), 4 * (NP - 1), _i32)
        vef[pl.ds(nkeep, 16)] = zeros

        m3 = iota < 3
        off3 = jnp.where(m3, iota, 0)
        r1 = jnp.where(m3, lax.rem(iota + 1, jnp.int32(3)), iota)
        r2 = jnp.where(m3, lax.rem(iota + 2, jnp.int32(3)), iota)
        nch = lax.shift_right_logical(nkeep + jnp.int32(15), 4)

        def seqc(c, carry):
            sl = pl.ds(c * 16, 16)
            rv4 = vrs[sl]
            cv4 = vcs[sl]
            fvv = vef[sl]
            for kk in range(16):
                lk = jnp.full((16,), kk, _i32)
                ib = rv4.at[lk].get(mode="promise_in_bounds")
                jb = cv4.at[lk].get(mode="promise_in_bounds")
                fb = fvv.at[lk].get(mode="promise_in_bounds")
                idxi = ib + off3
                idxj = jb + off3
                pi = plsc.load_gather(vpk, [idxi], mask=m3)
                pj = plsc.load_gather(vpk, [idxj], mask=m3)
                d = pi - pj
                t = d * d
                d2 = (t + t.at[r1].get(mode="promise_in_bounds")
                      + t.at[r2].get(mode="promise_in_bounds"))
                f = fb * _rsqrt2(d2)
                plsc.store_scatter(vpk, [idxi], pi + d * f, mask=m3)
            return carry
        lax.fori_loop(0, nch, seqc, 0)

        def c4_body(k, l2acc):
            sl = pl.ds(k * 16, 16)
            rv = vrow[sl]
            cv = vcol[sl]
            r4 = rv * 4
            c4_ = cv * 4
            xi = plsc.load_gather(vpk, [r4])
            yi = plsc.load_gather(vpk, [r4 + 1])
            zi = plsc.load_gather(vpk, [r4 + 2])
            xj = plsc.load_gather(vpk, [c4_])
            yj = plsc.load_gather(vpk, [c4_ + 1])
            zj = plsc.load_gather(vpk, [c4_ + 2])
            bx = xi - xj
            by = yi - yj
            bz = zi - zj
            d2 = bx * bx + by * by + bz * bz
            r = _rsqrt_n(d2, 2)
            cur = d2 * r
            ati = plsc.load_gather(vat, [rv])
            atj = plsc.load_gather(vat, [cv])
            tv = plsc.load_gather(vlt, [ati * 10 + atj])
            le = cur - tv
            l2acc = l2acc + le * le
            ratio = jnp.clip(tv * r, jnp.float32(0.98), jnp.float32(1.02))
            g = (ratio - jnp.float32(1.0)) * jnp.float32(0.005)
            plsc.addupdate_scatter(vax, [rv], bx * g, mask=mall)
            plsc.addupdate_scatter(vay, [rv], by * g, mask=mall)
            plsc.addupdate_scatter(vaz, [rv], bz * g, mask=mall)
            plsc.addupdate_scatter(vax, [cv], -(bx * g), mask=mall)
            plsc.addupdate_scatter(vay, [cv], -(by * g), mask=mall)
            plsc.addupdate_scatter(vaz, [cv], -(bz * g), mask=mall)
            return l2acc

        def c4(k2, l2acc):
            l2acc = c4_body(k2 * 2, l2acc)
            l2acc = c4_body(k2 * 2 + 1, l2acc)
            return l2acc
        l2acc = lax.fori_loop(0, EC // 2, c4, zeros)
        vtmp16[...] = l2acc
        pltpu.sync_copy(vtmp16, l2_h)

        iota4 = iota * 4
        def c5(a, carry):
            sl = pl.ds(a * 16, 16)
            base = jnp.full((16,), a * 64, _i32) + iota4
            vpx[sl] = plsc.load_gather(vpk, [base]) + vax[sl]
            vpy[sl] = plsc.load_gather(vpk, [base + 1]) + vay[sl]
            vpz[sl] = plsc.load_gather(vpk, [base + 2]) + vaz[sl]
            return carry
        lax.fori_loop(0, AC, c5, 0)

        pltpu.sync_copy(vpx, ox_h)
        pltpu.sync_copy(vpy, oy_h)
        pltpu.sync_copy(vpz, oz_h)
        pltpu.sync_copy(vrad, rad_h)


def _tc_body(xr_ref, yr_ref, zr_ref, rr_ref, xt_ref, yt_ref, zt_ref, rt_ref,
             l1_ref, l2_ref, ox_ref, oy_ref, oz_ref, loss_ref):
    b = pl.program_id(0)
    xi = xt_ref[...]
    yi = yt_ref[...]
    zi = zt_ref[...]
    xj = xr_ref[...]
    yj = yr_ref[...]
    zj = zr_ref[...]
    dx = xi - xj
    dy = yi - yj
    dz = zi - zj
    d2 = dx * dx + dy * dy + dz * dz
    rowid = b * 128 + lax.broadcasted_iota(_i32, (128, NP), 0)
    colid = lax.broadcasted_iota(_i32, (128, NP), 1)
    diag = rowid == colid
    valid = jnp.logical_not(diag) & (rowid < N) & (colid < N)
    dist = jnp.sqrt(jnp.where(diag, jnp.float32(1.0), d2))
    md = (rt_ref[...] + rr_ref[...]) * jnp.float32(0.8)
    gap = md - dist
    pen = jnp.where(valid, jnp.maximum(gap, jnp.float32(0.0)), jnp.float32(0.0))
    l3p = jnp.float32(0.5) * jnp.sum(pen * pen)
    cmask = valid & (dist < md) & (dist > jnp.float32(1e-8))
    w = jnp.where(cmask, gap * jnp.float32(0.0025) / dist, jnp.float32(0.0))
    s = jnp.sum(w, axis=1, keepdims=True)
    tx = jnp.sum(w * xj, axis=1, keepdims=True)
    ty = jnp.sum(w * yj, axis=1, keepdims=True)
    tz = jnp.sum(w * zj, axis=1, keepdims=True)
    ox_ref[...] = xi + (xi * s - tx)
    oy_ref[...] = yi + (yi * s - ty)
    oz_ref[...] = zi + (zi * s - tz)

    @pl.when(b == 0)
    def _():
        loss_ref[...] = jnp.reshape(
            jnp.sum(l1_ref[...]) + jnp.sum(l2_ref[...]) * jnp.float32(1.0 / 4000.0),
            (1, 1))

    loss_ref[...] = loss_ref[...] + jnp.reshape(l3p, (1, 1))

    @pl.when(b == NBLK - 1)
    def _():
        loss_ref[...] = loss_ref[...] * jnp.float32(0.1)


_tc_clash = pl.pallas_call(
    _tc_body,
    grid=(NBLK,),
    in_specs=[
        pl.BlockSpec((1, NP), lambda b: (0, 0)),
        pl.BlockSpec((1, NP), lambda b: (0, 0)),
        pl.BlockSpec((1, NP), lambda b: (0, 0)),
        pl.BlockSpec((1, NP), lambda b: (0, 0)),
        pl.BlockSpec((128, 1), lambda b: (b, 0)),
        pl.BlockSpec((128, 1), lambda b: (b, 0)),
        pl.BlockSpec((128, 1), lambda b: (b, 0)),
        pl.BlockSpec((128, 1), lambda b: (b, 0)),
        pl.BlockSpec((1, 16), lambda b: (0, 0)),
        pl.BlockSpec((1, 16), lambda b: (0, 0)),
    ],
    out_specs=[
        pl.BlockSpec((128, 1), lambda b: (b, 0)),
        pl.BlockSpec((128, 1), lambda b: (b, 0)),
        pl.BlockSpec((128, 1), lambda b: (b, 0)),
        pl.BlockSpec((1, 1), lambda b: (0, 0)),
    ],
    out_shape=[
        jax.ShapeDtypeStruct((NP, 1), _f32),
        jax.ShapeDtypeStruct((NP, 1), _f32),
        jax.ShapeDtypeStruct((NP, 1), _f32),
        jax.ShapeDtypeStruct((1, 1), _f32),
    ],
)


def kernel(pos, edge_index, atom_types):
    row = edge_index[0]
    col = edge_index[1]
    order = jnp.argsort(row).astype(_i32)
    pk = jnp.zeros((NP, 4), _f32).at[:N, :3].set(pos).reshape(-1)
    atp = jnp.zeros((NP,), _i32).at[:N].set(atom_types)
    ox, oy, oz, rad, l1, l2 = _sc_forward(
        pk, row, col, order, atp,
        jnp.asarray(_MV_T), jnp.asarray(_LT_T), jnp.asarray(_VDW_T))
    fx, fy, fz, loss = _tc_clash(
        ox.reshape(1, NP), oy.reshape(1, NP), oz.reshape(1, NP),
        rad.reshape(1, NP),
        ox.reshape(NP, 1), oy.reshape(NP, 1), oz.reshape(NP, 1),
        rad.reshape(NP, 1),
        l1.reshape(1, 16), l2.reshape(1, 16))
    pos_new = jnp.concatenate([fx, fy, fz], axis=1)[:N]
    return pos_new, loss[0, 0]
